# initial kernel scaffold (unmeasured)
import jax
import jax.numpy as jnp
from jax import lax
from jax.experimental import pallas as pl
from jax.experimental.pallas import tpu as pltpu

N_DEV = 4


def kernel(x, w_mat):
    m_glob, k_loc = x.shape
    n = w_mat.shape[1]
    m_per = m_glob // N_DEV

    def body(x_ref, w_ref, out_ref, comm_ref, send_sems, recv_sems):
        my = lax.axis_index("i")
        left = lax.rem(my + N_DEV - 1, N_DEV)
        right = lax.rem(my + 1, N_DEV)

        barrier_sem = pltpu.get_barrier_semaphore()
        for nbr in [left, right]:
            pl.semaphore_signal(
                barrier_sem, inc=1,
                device_id=(nbr,), device_id_type=pl.DeviceIdType.MESH,
            )
        pl.semaphore_wait(barrier_sem, 2)

        def partial(c):
            xc = x_ref[pl.ds(c * m_per, m_per), :]
            return jnp.dot(xc, w_ref[...], preferred_element_type=jnp.float32)

        comm_ref[0, :, :] = partial(lax.rem(my + N_DEV - 1, N_DEV))

        for h in range(N_DEV - 1):
            send_slot = h % 2
            recv_slot = (h + 1) % 2
            rdma = pltpu.make_async_remote_copy(
                src_ref=comm_ref.at[send_slot],
                dst_ref=comm_ref.at[recv_slot],
                send_sem=send_sems.at[send_slot],
                recv_sem=recv_sems.at[recv_slot],
                device_id=(right,),
                device_id_type=pl.DeviceIdType.MESH,
            )
            rdma.start()
            rdma.wait()

            c = lax.rem(my + 2 * N_DEV - 2 - h, N_DEV)
            comm_ref[recv_slot, :, :] = comm_ref[recv_slot, :, :] + partial(c)

        y = comm_ref[(N_DEV - 1) % 2, :, :]
        out_ref[...] = y * jax.nn.sigmoid(y)

    return pl.pallas_call(
        body,
        out_shape=jax.ShapeDtypeStruct((m_per, n), jnp.float32),
        in_specs=[
            pl.BlockSpec(memory_space=pltpu.VMEM),
            pl.BlockSpec(memory_space=pltpu.VMEM),
        ],
        out_specs=pl.BlockSpec(memory_space=pltpu.VMEM),
        scratch_shapes=[
            pltpu.VMEM((2, m_per, n), jnp.float32),
            pltpu.SemaphoreType.DMA((2,)),
            pltpu.SemaphoreType.DMA((2,)),
        ],
        compiler_params=pltpu.CompilerParams(collective_id=0),
    )(x, w_mat)


# baseline (device time: 190034 ns/iter reference)
import jax
import jax.numpy as jnp
from jax import lax
from jax.experimental import pallas as pl
from jax.experimental.pallas import tpu as pltpu

N_DEV = 4


def kernel(x, w_mat):
    m_glob, k_loc = x.shape
    n = w_mat.shape[1]
    m_per = m_glob // N_DEV

    x = x.astype(jnp.bfloat16)
    w_mat = w_mat.astype(jnp.bfloat16)

    def body(x_ref, w_ref, out_ref, comm_ref, send_sems, recv_sems):
        my = lax.axis_index("i")
        left = lax.rem(my + N_DEV - 1, N_DEV)
        right = lax.rem(my + 1, N_DEV)

        barrier_sem = pltpu.get_barrier_semaphore()
        for nbr in [left, right]:
            pl.semaphore_signal(
                barrier_sem, inc=1,
                device_id=(nbr,), device_id_type=pl.DeviceIdType.MESH,
            )
        pl.semaphore_wait(barrier_sem, 2)

        def partial(c):
            xc = x_ref[pl.ds(c * m_per, m_per), :]
            return jnp.dot(xc, w_ref[...], preferred_element_type=jnp.float32)

        comm_ref[0, :, :] = partial(lax.rem(my + N_DEV - 1, N_DEV)).astype(
            jnp.bfloat16
        )

        for h in range(N_DEV - 1):
            send_slot = h % 2
            recv_slot = (h + 1) % 2
            rdma = pltpu.make_async_remote_copy(
                src_ref=comm_ref.at[send_slot],
                dst_ref=comm_ref.at[recv_slot],
                send_sem=send_sems.at[send_slot],
                recv_sem=recv_sems.at[recv_slot],
                device_id=(right,),
                device_id_type=pl.DeviceIdType.MESH,
            )
            rdma.start()
            rdma.wait()

            c = lax.rem(my + 2 * N_DEV - 2 - h, N_DEV)
            acc = comm_ref[recv_slot, :, :].astype(jnp.float32) + partial(c)
            if h < N_DEV - 2:
                comm_ref[recv_slot, :, :] = acc.astype(jnp.bfloat16)
            else:
                out_ref[...] = acc * jax.nn.sigmoid(acc)

    return pl.pallas_call(
        body,
        out_shape=jax.ShapeDtypeStruct((m_per, n), jnp.float32),
        in_specs=[
            pl.BlockSpec(memory_space=pltpu.VMEM),
            pl.BlockSpec(memory_space=pltpu.VMEM),
        ],
        out_specs=pl.BlockSpec(memory_space=pltpu.VMEM),
        scratch_shapes=[
            pltpu.VMEM((2, m_per, n), jnp.bfloat16),
            pltpu.SemaphoreType.DMA((2,)),
            pltpu.SemaphoreType.DMA((2,)),
        ],
        compiler_params=pltpu.CompilerParams(
            collective_id=0,
            vmem_limit_bytes=44 * 1024 * 1024,
        ),
    )(x, w_mat)


# device time: 111744 ns/iter; 1.7006x vs baseline; 1.7006x over previous
import jax
import jax.numpy as jnp
from jax import lax
from jax.experimental import pallas as pl
from jax.experimental.pallas import tpu as pltpu

N_DEV = 4


def kernel(x, w_mat):
    m_glob, k_loc = x.shape
    n = w_mat.shape[1]
    m_per = m_glob // N_DEV
    n_half = n // 2

    x = x.astype(jnp.bfloat16)
    w_mat = w_mat.astype(jnp.bfloat16)

    def body(x_ref, w_ref, out_ref, rcomm, lcomm,
             rsend_sems, rrecv_sems, lsend_sems, lrecv_sems):
        my = lax.axis_index("i")
        left = lax.rem(my + N_DEV - 1, N_DEV)
        right = lax.rem(my + 1, N_DEV)

        barrier_sem = pltpu.get_barrier_semaphore()
        for nbr in [left, right]:
            pl.semaphore_signal(
                barrier_sem, inc=1,
                device_id=(nbr,), device_id_type=pl.DeviceIdType.MESH,
            )
        pl.semaphore_wait(barrier_sem, 2)

        def partial_r(c):
            xc = x_ref[pl.ds(c * m_per, m_per), :]
            return jnp.dot(xc, w_ref[:, :n_half],
                           preferred_element_type=jnp.float32)

        def partial_l(c):
            xc = x_ref[pl.ds(c * m_per, m_per), :]
            return jnp.dot(xc, w_ref[:, n_half:],
                           preferred_element_type=jnp.float32)

        rcomm[0, :, :] = partial_r(lax.rem(my + N_DEV - 1, N_DEV)).astype(
            jnp.bfloat16)
        lcomm[0, :, :] = partial_l(lax.rem(my + 1, N_DEV)).astype(
            jnp.bfloat16)

        for h in range(N_DEV - 1):
            ss = h % 2
            rs = (h + 1) % 2
            r_rdma = pltpu.make_async_remote_copy(
                src_ref=rcomm.at[ss],
                dst_ref=rcomm.at[rs],
                send_sem=rsend_sems.at[ss],
                recv_sem=rrecv_sems.at[rs],
                device_id=(right,),
                device_id_type=pl.DeviceIdType.MESH,
            )
            l_rdma = pltpu.make_async_remote_copy(
                src_ref=lcomm.at[ss],
                dst_ref=lcomm.at[rs],
                send_sem=lsend_sems.at[ss],
                recv_sem=lrecv_sems.at[rs],
                device_id=(left,),
                device_id_type=pl.DeviceIdType.MESH,
            )
            r_rdma.start()
            l_rdma.start()

            pr = partial_r(lax.rem(my + 2 * N_DEV - 2 - h, N_DEV))
            pll = partial_l(lax.rem(my + 2 + h, N_DEV))

            r_rdma.wait()
            acc_r = rcomm[rs, :, :].astype(jnp.float32) + pr
            if h < N_DEV - 2:
                rcomm[rs, :, :] = acc_r.astype(jnp.bfloat16)
            else:
                out_ref[:, :n_half] = acc_r * jax.nn.sigmoid(acc_r)

            l_rdma.wait()
            acc_l = lcomm[rs, :, :].astype(jnp.float32) + pll
            if h < N_DEV - 2:
                lcomm[rs, :, :] = acc_l.astype(jnp.bfloat16)
            else:
                out_ref[:, n_half:] = acc_l * jax.nn.sigmoid(acc_l)

    return pl.pallas_call(
        body,
        out_shape=jax.ShapeDtypeStruct((m_per, n), jnp.float32),
        in_specs=[
            pl.BlockSpec(memory_space=pltpu.VMEM),
            pl.BlockSpec(memory_space=pltpu.VMEM),
        ],
        out_specs=pl.BlockSpec(memory_space=pltpu.VMEM),
        scratch_shapes=[
            pltpu.VMEM((2, m_per, n_half), jnp.bfloat16),
            pltpu.VMEM((2, m_per, n_half), jnp.bfloat16),
            pltpu.SemaphoreType.DMA((2,)),
            pltpu.SemaphoreType.DMA((2,)),
            pltpu.SemaphoreType.DMA((2,)),
            pltpu.SemaphoreType.DMA((2,)),
        ],
        compiler_params=pltpu.CompilerParams(
            collective_id=0,
            vmem_limit_bytes=44 * 1024 * 1024,
        ),
    )(x, w_mat)


# device time: 103751 ns/iter; 1.8316x vs baseline; 1.0770x over previous
import jax
import jax.numpy as jnp
from jax import lax
from jax.experimental import pallas as pl
from jax.experimental.pallas import tpu as pltpu

N_DEV = 4
NSUB = 2


def kernel(x, w_mat):
    m_glob, k_loc = x.shape
    n = w_mat.shape[1]
    m_per = m_glob // N_DEV
    n_half = n // 2
    n_sub = n_half // NSUB

    x = x.astype(jnp.bfloat16)
    w_mat = w_mat.astype(jnp.bfloat16)

    def body(x_ref, w_ref, out_ref, rcomm, lcomm,
             rsend_sems, rrecv_sems, lsend_sems, lrecv_sems):
        my = lax.axis_index("i")
        left = lax.rem(my + N_DEV - 1, N_DEV)
        right = lax.rem(my + 1, N_DEV)

        barrier_sem = pltpu.get_barrier_semaphore()
        for nbr in [left, right]:
            pl.semaphore_signal(
                barrier_sem, inc=1,
                device_id=(nbr,), device_id_type=pl.DeviceIdType.MESH,
            )
        pl.semaphore_wait(barrier_sem, 2)

        def partial_r(c):
            xc = x_ref[pl.ds(c * m_per, m_per), :]
            return jnp.dot(xc, w_ref[:, :n_half],
                           preferred_element_type=jnp.float32)

        def partial_l(c):
            xc = x_ref[pl.ds(c * m_per, m_per), :]
            return jnp.dot(xc, w_ref[:, n_half:],
                           preferred_element_type=jnp.float32)

        def make(comm, send_sems, recv_sems, tgt, h, j):
            return pltpu.make_async_remote_copy(
                src_ref=comm.at[h % 2, j],
                dst_ref=comm.at[(h + 1) % 2, j],
                send_sem=send_sems.at[h % 2, j],
                recv_sem=recv_sems.at[(h + 1) % 2, j],
                device_id=(tgt,),
                device_id_type=pl.DeviceIdType.MESH,
            )

        r_rdma = [[make(rcomm, rsend_sems, rrecv_sems, right, h, j)
                   for j in range(NSUB)] for h in range(N_DEV - 1)]
        l_rdma = [[make(lcomm, lsend_sems, lrecv_sems, left, h, j)
                   for j in range(NSUB)] for h in range(N_DEV - 1)]

        p_r = partial_r(lax.rem(my + N_DEV - 1, N_DEV))
        p_l = partial_l(lax.rem(my + 1, N_DEV))
        for j in range(NSUB):
            cols = slice(j * n_sub, (j + 1) * n_sub)
            rcomm[0, j, :, :] = p_r[:, cols].astype(jnp.bfloat16)
            r_rdma[0][j].start()
            lcomm[0, j, :, :] = p_l[:, cols].astype(jnp.bfloat16)
            l_rdma[0][j].start()

        for h in range(N_DEV - 1):
            rs = (h + 1) % 2
            last = h == N_DEV - 2
            p_r = partial_r(lax.rem(my + 2 * N_DEV - 2 - h, N_DEV))
            p_l = partial_l(lax.rem(my + 2 + h, N_DEV))
            for j in range(NSUB):
                cols = slice(j * n_sub, (j + 1) * n_sub)
                if h >= 1:
                    r_rdma[h - 1][j].wait_send()
                r_rdma[h][j].wait_recv()
                acc = rcomm[rs, j, :, :].astype(jnp.float32) + p_r[:, cols]
                if not last:
                    rcomm[rs, j, :, :] = acc.astype(jnp.bfloat16)
                    r_rdma[h + 1][j].start()
                else:
                    out_ref[:, cols] = acc * jax.nn.sigmoid(acc)

                if h >= 1:
                    l_rdma[h - 1][j].wait_send()
                l_rdma[h][j].wait_recv()
                acc = lcomm[rs, j, :, :].astype(jnp.float32) + p_l[:, cols]
                if not last:
                    lcomm[rs, j, :, :] = acc.astype(jnp.bfloat16)
                    l_rdma[h + 1][j].start()
                else:
                    out_ref[:, n_half + j * n_sub:n_half + (j + 1) * n_sub] = (
                        acc * jax.nn.sigmoid(acc))

        for j in range(NSUB):
            r_rdma[N_DEV - 2][j].wait_send()
            l_rdma[N_DEV - 2][j].wait_send()

    return pl.pallas_call(
        body,
        out_shape=jax.ShapeDtypeStruct((m_per, n), jnp.float32),
        in_specs=[
            pl.BlockSpec(memory_space=pltpu.VMEM),
            pl.BlockSpec(memory_space=pltpu.VMEM),
        ],
        out_specs=pl.BlockSpec(memory_space=pltpu.VMEM),
        scratch_shapes=[
            pltpu.VMEM((2, NSUB, m_per, n_sub), jnp.bfloat16),
            pltpu.VMEM((2, NSUB, m_per, n_sub), jnp.bfloat16),
            pltpu.SemaphoreType.DMA((2, NSUB)),
            pltpu.SemaphoreType.DMA((2, NSUB)),
            pltpu.SemaphoreType.DMA((2, NSUB)),
            pltpu.SemaphoreType.DMA((2, NSUB)),
        ],
        compiler_params=pltpu.CompilerParams(
            collective_id=0,
            vmem_limit_bytes=44 * 1024 * 1024,
        ),
    )(x, w_mat)


# device time: 85840 ns/iter; 2.2138x vs baseline; 1.2087x over previous
import jax
import jax.numpy as jnp
from jax import lax
from jax.experimental import pallas as pl
from jax.experimental.pallas import tpu as pltpu

N_DEV = 4
NSUB = 4

_R_SLOT = (2, 1, 3)
_L_SLOT = (2, 0, 3)


def kernel(x, w_mat):
    m_glob, k_loc = x.shape
    n = w_mat.shape[1]
    m_per = m_glob // N_DEV
    n_half = n // 2
    n_sub = n_half // NSUB

    def body(x_hbm, w_ref, out_ref, wb_ref, xb_ref, fbuf, rcomm, lcomm,
             rsend_sems, rrecv_sems, lsend_sems, lrecv_sems, xsems):
        my = lax.axis_index("i")
        left = lax.rem(my + N_DEV - 1, N_DEV)
        right = lax.rem(my + 1, N_DEV)

        def xcopy(c, sem_i):
            return pltpu.make_async_copy(
                x_hbm.at[pl.ds(c * m_per, m_per), :],
                fbuf,
                xsems.at[sem_i],
            )

        cp = [
            xcopy(lax.rem(my + N_DEV - 1, N_DEV), 0),
            xcopy(lax.rem(my + 1, N_DEV), 1),
            xcopy(lax.rem(my + 2, N_DEV), 2),
            xcopy(my, 3),
        ]
        cp[0].start()

        wb_ref[...] = w_ref[...].astype(jnp.bfloat16)

        barrier_sem = pltpu.get_barrier_semaphore()
        for nbr in [left, right]:
            pl.semaphore_signal(
                barrier_sem, inc=1,
                device_id=(nbr,), device_id_type=pl.DeviceIdType.MESH,
            )
        pl.semaphore_wait(barrier_sem, 2)

        def make(comm, send_sems, recv_sems, tgt, h, j):
            return pltpu.make_async_remote_copy(
                src_ref=comm.at[h % 2, j],
                dst_ref=comm.at[(h + 1) % 2, j],
                send_sem=send_sems.at[h % 2, j],
                recv_sem=recv_sems.at[(h + 1) % 2, j],
                device_id=(tgt,),
                device_id_type=pl.DeviceIdType.MESH,
            )

        r_rdma = [[make(rcomm, rsend_sems, rrecv_sems, right, h, j)
                   for j in range(NSUB)] for h in range(N_DEV - 1)]
        l_rdma = [[make(lcomm, lsend_sems, lrecv_sems, left, h, j)
                   for j in range(NSUB)] for h in range(N_DEV - 1)]

        def rcols(j):
            return slice(j * n_sub, (j + 1) * n_sub)

        def lcols(j):
            return slice(n_half + j * n_sub, n_half + (j + 1) * n_sub)

        cp[0].wait()
        xb_ref[0] = fbuf[...].astype(jnp.bfloat16)
        cp[1].start()
        for j in range(NSUB):
            rcomm[0, j, :, :] = jnp.dot(
                xb_ref[0], wb_ref[:, rcols(j)],
                preferred_element_type=jnp.float32,
            ).astype(jnp.bfloat16)
            r_rdma[0][j].start()

        cp[1].wait()
        xb_ref[1] = fbuf[...].astype(jnp.bfloat16)
        cp[2].start()
        for j in range(NSUB):
            lcomm[0, j, :, :] = jnp.dot(
                xb_ref[1], wb_ref[:, lcols(j)],
                preferred_element_type=jnp.float32,
            ).astype(jnp.bfloat16)
            l_rdma[0][j].start()

        for h in range(N_DEV - 1):
            rs = (h + 1) % 2
            last = h == N_DEV - 2
            if h == 0:
                cp[2].wait()
                xb_ref[2] = fbuf[...].astype(jnp.bfloat16)
                cp[3].start()
            elif h == 1:
                cp[3].wait()
                xb_ref[3] = fbuf[...].astype(jnp.bfloat16)
            xc_r = xb_ref[_R_SLOT[h]]
            xc_l = xb_ref[_L_SLOT[h]]
            for j in range(NSUB):
                p_rj = jnp.dot(xc_r, wb_ref[:, rcols(j)],
                               preferred_element_type=jnp.float32)
                if h >= 1:
                    r_rdma[h - 1][j].wait_send()
                r_rdma[h][j].wait_recv()
                acc = rcomm[rs, j, :, :].astype(jnp.float32) + p_rj
                if not last:
                    rcomm[rs, j, :, :] = acc.astype(jnp.bfloat16)
                    r_rdma[h + 1][j].start()
                else:
                    out_ref[:, rcols(j)] = acc * jax.nn.sigmoid(acc)

                p_lj = jnp.dot(xc_l, wb_ref[:, lcols(j)],
                               preferred_element_type=jnp.float32)
                if h >= 1:
                    l_rdma[h - 1][j].wait_send()
                l_rdma[h][j].wait_recv()
                acc = lcomm[rs, j, :, :].astype(jnp.float32) + p_lj
                if not last:
                    lcomm[rs, j, :, :] = acc.astype(jnp.bfloat16)
                    l_rdma[h + 1][j].start()
                else:
                    out_ref[:, lcols(j)] = acc * jax.nn.sigmoid(acc)

        for j in range(NSUB):
            r_rdma[N_DEV - 2][j].wait_send()
            l_rdma[N_DEV - 2][j].wait_send()

    return pl.pallas_call(
        body,
        out_shape=jax.ShapeDtypeStruct((m_per, n), jnp.float32),
        in_specs=[
            pl.BlockSpec(memory_space=pltpu.MemorySpace.HBM),
            pl.BlockSpec(memory_space=pltpu.VMEM),
        ],
        out_specs=pl.BlockSpec(memory_space=pltpu.VMEM),
        scratch_shapes=[
            pltpu.VMEM((k_loc, n), jnp.bfloat16),
            pltpu.VMEM((N_DEV, m_per, k_loc), jnp.bfloat16),
            pltpu.VMEM((m_per, k_loc), jnp.float32),
            pltpu.VMEM((2, NSUB, m_per, n_sub), jnp.bfloat16),
            pltpu.VMEM((2, NSUB, m_per, n_sub), jnp.bfloat16),
            pltpu.SemaphoreType.DMA((2, NSUB)),
            pltpu.SemaphoreType.DMA((2, NSUB)),
            pltpu.SemaphoreType.DMA((2, NSUB)),
            pltpu.SemaphoreType.DMA((2, NSUB)),
            pltpu.SemaphoreType.DMA((4,)),
        ],
        compiler_params=pltpu.CompilerParams(
            collective_id=0,
            vmem_limit_bytes=44 * 1024 * 1024,
        ),
    )(x, w_mat)
